# baseline (speedup 1.0000x reference)
import functools

import jax
import jax.numpy as jnp
from jax import lax
from jax.experimental import pallas as pl
from jax.experimental.pallas import tpu as pltpu
from jax.experimental.pallas import tpu_sc as plsc

N = 10000
E = 160000
NFEAT = 256
NHID = 256
NCLASS = 64

NC = 2
NS = 16
CH = 128

NP = 10112
RPT = NP // NS

NCH1 = 80
EPT1 = NCH1 * CH

NCH2 = 40
EPT2 = NCH2 * CH

BS = 1000
NBLK = N // BS

_mesh = plsc.VectorSubcoreMesh(
    core_axis_name="c", subcore_axis_name="s", num_cores=NC, num_subcores=NS
)

_RCHUNKS = [(i * CH, min(CH, RPT - i * CH)) for i in range((RPT + CH - 1) // CH)]


def _zero_spmem(rows, acc, rbase, sem):
    for off, n in _RCHUNKS:
        pltpu.async_copy(rows.at[pl.ds(0, n)], acc.at[pl.ds(rbase + off, n)], sem)
    for off, n in _RCHUNKS:
        pltpu.make_async_copy(rows.at[pl.ds(0, n)], acc.at[pl.ds(rbase + off, n)], sem).wait()


def _writeout_spmem(rows_a, rows_b, acc, rbase, out_slice_fn, sem_a, sem_b):
    bufs = [(rows_a, sem_a), (rows_b, sem_b)]

    def sp2vm(k, issue):
        off, n = _RCHUNKS[k]
        buf, sem = bufs[k % 2]
        cp = pltpu.async_copy if issue else pltpu.make_async_copy
        r = cp(acc.at[pl.ds(rbase + off, n)], buf.at[pl.ds(0, n)], sem)
        if not issue:
            r.wait()

    def push(k, issue):
        off, n = _RCHUNKS[k]
        buf, sem = bufs[k % 2]
        cp = pltpu.async_copy if issue else pltpu.make_async_copy
        r = cp(buf.at[pl.ds(0, n)], out_slice_fn(rbase + off, n), sem)
        if not issue:
            r.wait()

    n_ch = len(_RCHUNKS)
    sp2vm(0, True)
    sp2vm(1, True)
    for k in range(n_ch):
        if k >= 2:
            push(k - 2, False)
            sp2vm(k, True)
        sp2vm(k, False)
        push(k, True)
    push(n_ch - 2, False)
    push(n_ch - 1, False)


def _seg_loop(table, edges, w, nch, rows, esidx, gsem, ssem, isem, acc,
              deg_scatter=None):
    def g_issue(q, r):
        pltpu.async_copy(table.at[esidx.at[q, 0]], rows[r], gsem[r])

    def g_wait(r):
        pltpu.make_async_copy(table.at[esidx.at[0, 0]], rows[r], gsem[r]).wait()

    def s_issue(q, r):
        pltpu.async_copy(rows[r], acc.at[esidx.at[q, 1]], ssem[r], add=True)
        if deg_scatter is not None:
            ones_r, dacc, dsem = deg_scatter
            pltpu.async_copy(ones_r, dacc.at[esidx.at[q, 1]], dsem[r], add=True)

    def s_wait(r):
        pltpu.make_async_copy(rows[r], acc.at[esidx.at[0, 1]], ssem[r]).wait()
        if deg_scatter is not None:
            ones_r, dacc, dsem = deg_scatter
            pltpu.make_async_copy(ones_r, dacc.at[esidx.at[0, 1]], dsem[r]).wait()

    def i_issue(j, q):
        jc = jnp.where(j < nch, j, 0)
        pltpu.async_copy(edges.at[w, jc], esidx.at[q], isem[q])

    def i_wait(q):
        pltpu.make_async_copy(edges.at[w, 0], esidx.at[q], isem[q]).wait()

    pltpu.sync_copy(edges.at[w, 0], esidx.at[0])
    g_issue(0, 0)
    i_issue(1, 1)
    i_issue(2, 2)

    def body(i, carry):
        j_base = i * 4
        for t in range(4):
            j = j_base + t
            r = t % 2
            nr = (t + 1) % 2
            nq = (t + 1) % 4
            pq = (t + 3) % 4
            g_wait(r)
            s_issue(t, r)
            if t == 0:
                @pl.when(i > 0)
                def _():
                    s_wait(nr)
            else:
                s_wait(nr)
            i_issue(j + 3, pq)
            i_wait(nq)
            g_issue(nq, nr)
        return carry

    lax.fori_loop(0, nch // 4, body, 0)
    g_wait(0)
    s_wait(1)
    i_wait(1)
    i_wait(2)


def _sc1_body(table, edges, zrows, zdeg, ones1,
              agg_out, deg_out,
              esidx, rows_a, rows_b, ones_r, degb, acc, dacc,
              g0, g1, s0, s1, d0, d1, i0, i1, i2, i3):
    c = lax.axis_index("c")
    s = lax.axis_index("s")
    rbase = pl.multiple_of(s * RPT, 8)
    w = c * NS + s
    pltpu.sync_copy(zrows, rows_b)
    _zero_spmem(rows_b, acc, rbase, g1)
    pltpu.sync_copy(zdeg, degb)
    pltpu.sync_copy(degb, dacc.at[pl.ds(rbase, RPT)])
    pltpu.sync_copy(ones1, ones_r)
    plsc.subcore_barrier()

    _seg_loop(table, edges, w, NCH1,
              [rows_a, rows_b], esidx,
              [g0, g1], [s0, s1], [i0, i1, i2, i3],
              acc, deg_scatter=(ones_r, dacc, [d0, d1]))
    plsc.subcore_barrier()

    _writeout_spmem(rows_a, rows_b, acc, rbase,
                    lambda r, n: agg_out.at[c, pl.ds(r, n)], g0, g1)

    @pl.when(c == 0)
    def _():
        pltpu.sync_copy(dacc.at[pl.ds(rbase, RPT)], degb)
        pltpu.sync_copy(degb, deg_out.at[pl.ds(rbase, RPT)])


@functools.cache
def _sc1():
  return pl.kernel(
    _sc1_body,
    out_type=(
        jax.ShapeDtypeStruct((NC, NP, 128), jnp.float32),
        jax.ShapeDtypeStruct((NP,), jnp.float32),
    ),
    mesh=_mesh,
    scratch_types=[
        pltpu.VMEM((4, 2, CH), jnp.int32),
        pltpu.VMEM((CH, 128), jnp.float32),
        pltpu.VMEM((CH, 128), jnp.float32),
        pltpu.VMEM((CH,), jnp.float32),
        pltpu.VMEM((RPT,), jnp.float32),
        pltpu.VMEM_SHARED((NP, 128), jnp.float32),
        pltpu.VMEM_SHARED((NP,), jnp.float32),
    ] + [pltpu.SemaphoreType.DMA] * 10,
  )


def _sc2_body(table, edges, zrows,
              agg_out,
              esidx, rows_a, rows_b, acc,
              g0, g1, s0, s1, i0, i1, i2, i3):
    c = lax.axis_index("c")
    s = lax.axis_index("s")
    rbase = pl.multiple_of(s * RPT, 8)
    w = c * NS + s
    pltpu.sync_copy(zrows, rows_b)
    _zero_spmem(rows_b, acc, rbase, g1)
    plsc.subcore_barrier()

    _seg_loop(table, edges, w, NCH2,
              [rows_a, rows_b], esidx,
              [g0, g1], [s0, s1], [i0, i1, i2, i3],
              acc)
    plsc.subcore_barrier()

    _writeout_spmem(rows_a, rows_b, acc, rbase,
                    lambda r, n: agg_out.at[c, pl.ds(r, n)], g0, g1)


@functools.cache
def _sc2():
  return pl.kernel(
    _sc2_body,
    out_type=jax.ShapeDtypeStruct((NC, NP, 128), jnp.float32),
    mesh=_mesh,
    scratch_types=[
        pltpu.VMEM((4, 2, CH), jnp.int32),
        pltpu.VMEM((CH, 128), jnp.float32),
        pltpu.VMEM((CH, 128), jnp.float32),
        pltpu.VMEM_SHARED((NP, 128), jnp.float32),
    ] + [pltpu.SemaphoreType.DMA] * 8,
  )


def _k1_body(x_ref, w_ref, out_ref):
    out_ref[...] = jnp.dot(
        x_ref[...], w_ref[...], preferred_element_type=jnp.float32
    )[None]


def _k1(x, W1):
    return pl.pallas_call(
        _k1_body,
        grid=(8, NBLK),
        in_specs=[
            pl.BlockSpec((BS, NFEAT), lambda c, nb: (nb, 0)),
            pl.BlockSpec((NFEAT, 128), lambda c, nb: (0, c // 4)),
        ],
        out_specs=pl.BlockSpec((1, BS, 128), lambda c, nb: (c, nb, 0)),
        out_shape=jax.ShapeDtypeStruct((8, N, 128), jnp.float32),
    )(x, W1)


def _k2_body(a0_ref, a1_ref, s0_ref, s1_ref, deg_ref, b1_ref, w2_ref, out_ref):
    inv = 1.0 / (deg_ref[...] + 1.0)
    h0 = jnp.maximum((a0_ref[0] + s0_ref[0]) * inv + b1_ref[:, :128], 0.0)
    h1 = jnp.maximum((a1_ref[0] + s1_ref[0]) * inv + b1_ref[:, 128:], 0.0)
    h = jnp.concatenate([h0, h1], axis=1)
    out_ref[...] = jnp.dot(
        h, w2_ref[...], preferred_element_type=jnp.float32
    )[None]


def _k2(agg1, sup1, deg, b1, W2p):
    return pl.pallas_call(
        _k2_body,
        grid=(8, NBLK),
        in_specs=[
            pl.BlockSpec((1, BS, 128), lambda c, nb: (0, nb, 0)),
            pl.BlockSpec((1, BS, 128), lambda c, nb: (1, nb, 0)),
            pl.BlockSpec((1, BS, 128), lambda c, nb: (0, nb, 0)),
            pl.BlockSpec((1, BS, 128), lambda c, nb: (4, nb, 0)),
            pl.BlockSpec((BS, 1), lambda c, nb: (nb, 0)),
            pl.BlockSpec((1, NHID), lambda c, nb: (0, 0)),
            pl.BlockSpec((NHID, 128), lambda c, nb: (0, 0)),
        ],
        out_specs=pl.BlockSpec((1, BS, 128), lambda c, nb: (c, nb, 0)),
        out_shape=jax.ShapeDtypeStruct((8, N, 128), jnp.float32),
    )(agg1, agg1, sup1, sup1, deg, b1, W2p)


def _k3_body(p0_ref, p1_ref, s_ref, deg_ref, b2_ref, out_ref):
    inv = 1.0 / (deg_ref[...] + 1.0)
    o = (p0_ref[0, :, :64] + p1_ref[0, :, :64] + s_ref[:, :64]) * inv + b2_ref[...]
    m = jnp.max(o, axis=1, keepdims=True)
    e = o - m
    lse = jnp.log(jnp.sum(jnp.exp(e), axis=1, keepdims=True))
    out_ref[...] = e - lse


def _k3(agg2, sup2, deg, b2):
    return pl.pallas_call(
        _k3_body,
        grid=(NBLK,),
        in_specs=[
            pl.BlockSpec((1, BS, 128), lambda nb: (0, nb, 0)),
            pl.BlockSpec((1, BS, 128), lambda nb: (1, nb, 0)),
            pl.BlockSpec((BS, 128), lambda nb: (nb, 0)),
            pl.BlockSpec((BS, 1), lambda nb: (nb, 0)),
            pl.BlockSpec((1, NCLASS), lambda nb: (0, 0)),
        ],
        out_specs=pl.BlockSpec((BS, NCLASS), lambda nb: (nb, 0)),
        out_shape=jax.ShapeDtypeStruct((N, NCLASS), jnp.float32),
    )(agg2, agg2, sup2, deg, b2)


def _pad_edges(arr, n_parts, ept, fill):
    per = E // n_parts
    a = arr.reshape(n_parts, per)
    return jnp.pad(a, ((0, 0), (0, ept - per)), constant_values=fill)


@jax.jit
def kernel(x, adj, W1, b1, W2, b2):
    src = adj[0].astype(jnp.int32)
    dst = adj[1].astype(jnp.int32)

    wids = jnp.arange(NC * NS, dtype=jnp.int32)
    toff = ((wids // NS * 4 + wids % NS // 4) * N).reshape(-1, 1, 1, 1)

    src1 = _pad_edges(src, NS, EPT1, 0).reshape(NS, NCH1, 1, CH)
    dst1 = _pad_edges(dst, NS, EPT1, N).reshape(NS, NCH1, 1, CH)
    src1 = jnp.concatenate([src1, src1], axis=0) + toff
    dst1 = jnp.concatenate([dst1, dst1], axis=0)
    edges1 = jnp.concatenate([src1, dst1], axis=2)
    src2 = _pad_edges(src, NC * NS, EPT2, 0).reshape(NC * NS, NCH2, 1, CH)
    src2 = src2 + toff
    dst2 = _pad_edges(dst, NC * NS, EPT2, N).reshape(NC * NS, NCH2, 1, CH)
    edges2 = jnp.concatenate([src2, dst2], axis=2)

    zrows = jnp.zeros((CH, 128), jnp.float32)
    zdeg = jnp.zeros((RPT,), jnp.float32)
    ones1 = jnp.ones((CH,), jnp.float32)

    sup1 = _k1(x, W1)
    table1 = sup1.reshape(8 * N, 128)
    agg1, deg = _sc1()(table1, edges1, zrows, zdeg, ones1)
    deg_col = deg[:N].reshape(N, 1)
    b1r = b1.reshape(1, NHID)

    W2p = jnp.pad(W2, ((0, 0), (0, 128 - NCLASS)))
    sup2 = _k2(agg1, sup1, deg_col, b1r, W2p)
    table2 = sup2.reshape(8 * N, 128)
    agg2 = _sc2()(table2, edges2, zrows)
    return _k3(agg2, sup2[0], deg_col, b2.reshape(1, NCLASS))


# speedup vs baseline: 1.1095x; 1.1095x over previous
import functools

import jax
import jax.numpy as jnp
from jax import lax
from jax.experimental import pallas as pl
from jax.experimental.pallas import tpu as pltpu
from jax.experimental.pallas import tpu_sc as plsc

N = 10000
E = 160000
NFEAT = 256
NHID = 256
NCLASS = 64

NC = 2
NS = 16
CH = 128

NP = 10112
RPT = NP // NS

NCH1 = 80
EPT1 = NCH1 * CH

NCH2 = 40
EPT2 = NCH2 * CH

BS = 1000
NBLK = N // BS

_mesh = plsc.VectorSubcoreMesh(
    core_axis_name="c", subcore_axis_name="s", num_cores=NC, num_subcores=NS
)

_RCHUNKS = [(i * CH, min(CH, RPT - i * CH)) for i in range((RPT + CH - 1) // CH)]


def _zero_spmem(rows, acc, rbase, sem):
    for off, n in _RCHUNKS:
        pltpu.async_copy(rows.at[pl.ds(0, n)], acc.at[pl.ds(rbase + off, n)], sem)
    for off, n in _RCHUNKS:
        pltpu.make_async_copy(rows.at[pl.ds(0, n)], acc.at[pl.ds(rbase + off, n)], sem).wait()


def _writeout_spmem(rows_a, rows_b, acc, rbase, out_slice_fn, sem_a, sem_b):
    bufs = [(rows_a, sem_a), (rows_b, sem_b)]

    def sp2vm(k, issue):
        off, n = _RCHUNKS[k]
        buf, sem = bufs[k % 2]
        cp = pltpu.async_copy if issue else pltpu.make_async_copy
        r = cp(acc.at[pl.ds(rbase + off, n)], buf.at[pl.ds(0, n)], sem)
        if not issue:
            r.wait()

    def push(k, issue):
        off, n = _RCHUNKS[k]
        buf, sem = bufs[k % 2]
        cp = pltpu.async_copy if issue else pltpu.make_async_copy
        r = cp(buf.at[pl.ds(0, n)], out_slice_fn(rbase + off, n), sem)
        if not issue:
            r.wait()

    n_ch = len(_RCHUNKS)
    sp2vm(0, True)
    sp2vm(1, True)
    for k in range(n_ch):
        if k >= 2:
            push(k - 2, False)
            sp2vm(k, True)
        sp2vm(k, False)
        push(k, True)
    push(n_ch - 2, False)
    push(n_ch - 1, False)


def _seg_loop(table, edges, w, nch, rows, esidx, gsem, ssem, isem, acc,
              deg_scatter=None):
    def g_issue(q, r):
        pltpu.async_copy(table.at[esidx.at[q, 0]], rows[r], gsem[r])

    def g_wait(r):
        pltpu.make_async_copy(table.at[esidx.at[0, 0]], rows[r], gsem[r]).wait()

    def s_issue(q, r):
        pltpu.async_copy(rows[r], acc.at[esidx.at[q, 1]], ssem[r], add=True)
        if deg_scatter is not None:
            ones_r, dacc, dsem = deg_scatter
            pltpu.async_copy(ones_r, dacc.at[esidx.at[q, 1]], dsem[r], add=True)

    def s_wait(r):
        pltpu.make_async_copy(rows[r], acc.at[esidx.at[0, 1]], ssem[r]).wait()
        if deg_scatter is not None:
            ones_r, dacc, dsem = deg_scatter
            pltpu.make_async_copy(ones_r, dacc.at[esidx.at[0, 1]], dsem[r]).wait()

    def i_issue(j, q):
        jc = jnp.where(j < nch, j, 0)
        pltpu.async_copy(edges.at[w, jc], esidx.at[q], isem[q])

    def i_wait(q):
        pltpu.make_async_copy(edges.at[w, 0], esidx.at[q], isem[q]).wait()

    pltpu.sync_copy(edges.at[w, 0], esidx.at[0])
    g_issue(0, 0)
    i_issue(1, 1)
    i_issue(2, 2)

    def body(i, carry):
        j_base = i * 4
        for t in range(4):
            j = j_base + t
            r = t % 2
            nr = (t + 1) % 2
            nq = (t + 1) % 4
            pq = (t + 3) % 4
            g_wait(r)
            s_issue(t, r)
            if t == 0:
                @pl.when(i > 0)
                def _():
                    s_wait(nr)
            else:
                s_wait(nr)
            i_issue(j + 3, pq)
            i_wait(nq)
            g_issue(nq, nr)
        return carry

    lax.fori_loop(0, nch // 4, body, 0)
    g_wait(0)
    s_wait(1)
    i_wait(1)
    i_wait(2)


def _sc1_body(table, edges, zrows, zdeg, ones1,
              agg_out, deg_out,
              esidx, rows_a, rows_b, ones_r, degb, acc, dacc,
              g0, g1, s0, s1, d0, d1, i0, i1, i2, i3):
    c = lax.axis_index("c")
    s = lax.axis_index("s")
    rbase = pl.multiple_of(s * RPT, 8)
    w = c * NS + s
    pltpu.sync_copy(zrows, rows_b)
    _zero_spmem(rows_b, acc, rbase, g1)
    pltpu.sync_copy(zdeg, degb)
    pltpu.sync_copy(degb, dacc.at[pl.ds(rbase, RPT)])
    pltpu.sync_copy(ones1, ones_r)
    plsc.subcore_barrier()

    _seg_loop(table, edges, w, NCH1,
              [rows_a, rows_b], esidx,
              [g0, g1], [s0, s1], [i0, i1, i2, i3],
              acc, deg_scatter=(ones_r, dacc, [d0, d1]))
    plsc.subcore_barrier()

    _writeout_spmem(rows_a, rows_b, acc, rbase,
                    lambda r, n: agg_out.at[c, pl.ds(r, n)], g0, g1)

    @pl.when(c == 0)
    def _():
        pltpu.sync_copy(dacc.at[pl.ds(rbase, RPT)], degb)
        pltpu.sync_copy(degb, deg_out.at[pl.ds(rbase, RPT)])


@functools.cache
def _sc1():
  return pl.kernel(
    _sc1_body,
    out_type=(
        jax.ShapeDtypeStruct((NC, NP, 128), jnp.float32),
        jax.ShapeDtypeStruct((NP,), jnp.float32),
    ),
    mesh=_mesh,
    scratch_types=[
        pltpu.VMEM((4, 2, CH), jnp.int32),
        pltpu.VMEM((CH, 128), jnp.float32),
        pltpu.VMEM((CH, 128), jnp.float32),
        pltpu.VMEM((CH,), jnp.float32),
        pltpu.VMEM((RPT,), jnp.float32),
        pltpu.VMEM_SHARED((NP, 128), jnp.float32),
        pltpu.VMEM_SHARED((NP,), jnp.float32),
    ] + [pltpu.SemaphoreType.DMA] * 10,
  )


def _sc2_body(table, edges, zrows,
              agg_out,
              esidx, rows_a, rows_b, acc,
              g0, g1, s0, s1, i0, i1, i2, i3):
    c = lax.axis_index("c")
    s = lax.axis_index("s")
    rbase = pl.multiple_of(s * RPT, 8)
    w = c * NS + s
    pltpu.sync_copy(zrows, rows_b)
    _zero_spmem(rows_b, acc, rbase, g1)
    plsc.subcore_barrier()

    _seg_loop(table, edges, w, NCH2,
              [rows_a, rows_b], esidx,
              [g0, g1], [s0, s1], [i0, i1, i2, i3],
              acc)
    plsc.subcore_barrier()

    _writeout_spmem(rows_a, rows_b, acc, rbase,
                    lambda r, n: agg_out.at[c, pl.ds(r, n)], g0, g1)


@functools.cache
def _sc2():
  return pl.kernel(
    _sc2_body,
    out_type=jax.ShapeDtypeStruct((NC, NP, 128), jnp.float32),
    mesh=_mesh,
    scratch_types=[
        pltpu.VMEM((4, 2, CH), jnp.int32),
        pltpu.VMEM((CH, 128), jnp.float32),
        pltpu.VMEM((CH, 128), jnp.float32),
        pltpu.VMEM_SHARED((NP, 128), jnp.float32),
    ] + [pltpu.SemaphoreType.DMA] * 8,
  )


def _k1_body(x_ref, w_ref, out_ref):
    out_ref[...] = jnp.dot(
        x_ref[...], w_ref[...], preferred_element_type=jnp.float32
    )[None]


def _k1(x, W1):
    return pl.pallas_call(
        _k1_body,
        grid=(4, NBLK),
        in_specs=[
            pl.BlockSpec((BS, NFEAT), lambda c, nb: (nb, 0)),
            pl.BlockSpec((NFEAT, 128), lambda c, nb: (0, c // 2)),
        ],
        out_specs=pl.BlockSpec((1, BS, 128), lambda c, nb: (c, nb, 0)),
        out_shape=jax.ShapeDtypeStruct((4, N, 128), jnp.float32),
    )(x, W1)


def _k2_body(a0_ref, a1_ref, s0_ref, s1_ref, deg_ref, b1_ref, w2_ref, out_ref):
    inv = 1.0 / (deg_ref[...] + 1.0)
    h0 = jnp.maximum((a0_ref[0] + s0_ref[0]) * inv + b1_ref[:, :128], 0.0)
    h1 = jnp.maximum((a1_ref[0] + s1_ref[0]) * inv + b1_ref[:, 128:], 0.0)
    h = jnp.concatenate([h0, h1], axis=1)
    out_ref[...] = jnp.dot(
        h, w2_ref[...], preferred_element_type=jnp.float32
    )[None]


def _k2(agg1, sup1, deg, b1, W2p):
    return pl.pallas_call(
        _k2_body,
        grid=(4, NBLK),
        in_specs=[
            pl.BlockSpec((1, BS, 128), lambda c, nb: (0, nb, 0)),
            pl.BlockSpec((1, BS, 128), lambda c, nb: (1, nb, 0)),
            pl.BlockSpec((1, BS, 128), lambda c, nb: (0, nb, 0)),
            pl.BlockSpec((1, BS, 128), lambda c, nb: (2, nb, 0)),
            pl.BlockSpec((BS, 1), lambda c, nb: (nb, 0)),
            pl.BlockSpec((1, NHID), lambda c, nb: (0, 0)),
            pl.BlockSpec((NHID, 128), lambda c, nb: (0, 0)),
        ],
        out_specs=pl.BlockSpec((1, BS, 128), lambda c, nb: (c, nb, 0)),
        out_shape=jax.ShapeDtypeStruct((4, N, 128), jnp.float32),
    )(agg1, agg1, sup1, sup1, deg, b1, W2p)


def _k3_body(p0_ref, p1_ref, s_ref, deg_ref, b2_ref, out_ref):
    inv = 1.0 / (deg_ref[...] + 1.0)
    o = (p0_ref[0, :, :64] + p1_ref[0, :, :64] + s_ref[:, :64]) * inv + b2_ref[...]
    m = jnp.max(o, axis=1, keepdims=True)
    e = o - m
    lse = jnp.log(jnp.sum(jnp.exp(e), axis=1, keepdims=True))
    out_ref[...] = e - lse


def _k3(agg2, sup2, deg, b2):
    return pl.pallas_call(
        _k3_body,
        grid=(NBLK,),
        in_specs=[
            pl.BlockSpec((1, BS, 128), lambda nb: (0, nb, 0)),
            pl.BlockSpec((1, BS, 128), lambda nb: (1, nb, 0)),
            pl.BlockSpec((BS, 128), lambda nb: (nb, 0)),
            pl.BlockSpec((BS, 1), lambda nb: (nb, 0)),
            pl.BlockSpec((1, NCLASS), lambda nb: (0, 0)),
        ],
        out_specs=pl.BlockSpec((BS, NCLASS), lambda nb: (nb, 0)),
        out_shape=jax.ShapeDtypeStruct((N, NCLASS), jnp.float32),
    )(agg2, agg2, sup2, deg, b2)


def _pad_edges(arr, n_parts, ept, fill):
    per = E // n_parts
    a = arr.reshape(n_parts, per)
    return jnp.pad(a, ((0, 0), (0, ept - per)), constant_values=fill)


@jax.jit
def kernel(x, adj, W1, b1, W2, b2):
    src = adj[0].astype(jnp.int32)
    dst = adj[1].astype(jnp.int32)

    wids = jnp.arange(NC * NS, dtype=jnp.int32)
    toff = ((wids // NS * 2 + wids % NS // 8) * N).reshape(-1, 1, 1, 1)

    src1 = _pad_edges(src, NS, EPT1, 0).reshape(NS, NCH1, 1, CH)
    dst1 = _pad_edges(dst, NS, EPT1, N).reshape(NS, NCH1, 1, CH)
    src1 = jnp.concatenate([src1, src1], axis=0) + toff
    dst1 = jnp.concatenate([dst1, dst1], axis=0)
    edges1 = jnp.concatenate([src1, dst1], axis=2)
    src2 = _pad_edges(src, NC * NS, EPT2, 0).reshape(NC * NS, NCH2, 1, CH)
    src2 = src2 + toff
    dst2 = _pad_edges(dst, NC * NS, EPT2, N).reshape(NC * NS, NCH2, 1, CH)
    edges2 = jnp.concatenate([src2, dst2], axis=2)

    zrows = jnp.zeros((CH, 128), jnp.float32)
    zdeg = jnp.zeros((RPT,), jnp.float32)
    ones1 = jnp.ones((CH,), jnp.float32)

    sup1 = _k1(x, W1)
    table1 = sup1.reshape(4 * N, 128)
    agg1, deg = _sc1()(table1, edges1, zrows, zdeg, ones1)
    deg_col = deg[:N].reshape(N, 1)
    b1r = b1.reshape(1, NHID)

    W2p = jnp.pad(W2, ((0, 0), (0, 128 - NCLASS)))
    sup2 = _k2(agg1, sup1, deg_col, b1r, W2p)
    table2 = sup2.reshape(4 * N, 128)
    agg2 = _sc2()(table2, edges2, zrows)
    return _k3(agg2, sup2[0], deg_col, b2.reshape(1, NCLASS))
